# baseline (speedup 1.0000x reference)
import jax
import jax.numpy as jnp
from jax.experimental import pallas as pl
from jax.experimental.pallas import tpu as pltpu

FEAT_SIZE = 38.0
THRESHOLD = 0.5

_B = 16
_HW5 = 7220
_NC = 80
_CH = 722
_NB = _HW5 // _CH
_G = _B * _NB


def _body(cs_ref, conf_ref, box_ref, prior_ref, boxo_ref, probs_ref, idx_ref):
    cs = cs_ref[...]
    conf = conf_ref[...]
    scores = cs * conf
    m = jnp.max(scores, axis=-1)
    iota = jax.lax.broadcasted_iota(jnp.int32, scores.shape, 2)
    am = jnp.min(jnp.where(scores == m[..., None], iota, _NC), axis=-1)
    mask = m > THRESHOLD

    box = box_ref[...]
    prior = prior_ref[...]
    xy = box[..., :2] + prior[..., :2]
    wh = box[..., 2:] * prior[..., 2:]
    mins = xy - wh / 2.0
    maxs = xy + wh / 2.0
    corners = jnp.concatenate([mins, maxs], axis=-1) / FEAT_SIZE
    boxo_ref[...] = jnp.where(mask[..., None], corners, 0.0)
    probs_ref[...] = jnp.where(mask, m, 0.0)[:, None, :]
    idx_ref[...] = am[:, None, :]


def kernel(box, box_confidence, class_score, prior):
    cs = class_score.reshape(_G, _CH, _NC)
    conf = box_confidence.reshape(_G, _CH, 1)
    boxr = box.reshape(_G, _CH, 4)
    priorr = prior.reshape(_NB, _CH, 4)

    boxo, probs, idx = pl.pallas_call(
        _body,
        grid=(_G,),
        in_specs=[
            pl.BlockSpec((1, _CH, _NC), lambda g: (g, 0, 0)),
            pl.BlockSpec((1, _CH, 1), lambda g: (g, 0, 0)),
            pl.BlockSpec((1, _CH, 4), lambda g: (g, 0, 0)),
            pl.BlockSpec((1, _CH, 4), lambda g: (g % _NB, 0, 0)),
        ],
        out_specs=[
            pl.BlockSpec((1, _CH, 4), lambda g: (g, 0, 0)),
            pl.BlockSpec((1, 1, _CH), lambda g: (g, 0, 0)),
            pl.BlockSpec((1, 1, _CH), lambda g: (g, 0, 0)),
        ],
        out_shape=[
            jax.ShapeDtypeStruct((_G, _CH, 4), jnp.float32),
            jax.ShapeDtypeStruct((_G, 1, _CH), jnp.float32),
            jax.ShapeDtypeStruct((_G, 1, _CH), jnp.int32),
        ],
        compiler_params=pltpu.CompilerParams(
            dimension_semantics=("parallel",),
        ),
    )(cs, conf, boxr, priorr)

    box_out = boxo.reshape(16, 1444, 5, 4)
    probs_out = probs.reshape(16, 1444, 5)
    idx_out = idx.reshape(16, 1444, 5)
    return box_out, probs_out, idx_out
